# Initial kernel scaffold; baseline (speedup 1.0000x reference)
import jax
import jax.numpy as jnp
from jax.experimental import pallas as pl


def kernel(x, edge_index, Wl0, Wr0, b0, Wl1, Wr1, b1, Wl2, Wr2, b2):
    raise NotImplementedError("write your pallas kernel here")


# speedup vs baseline: 4.0259x; 4.0259x over previous
import functools

import jax
import jax.numpy as jnp
from jax import lax
from jax.experimental import pallas as pl
from jax.experimental.pallas import tpu as pltpu
from jax.experimental.pallas import tpu_sc as plsc

N = 10000
E = 320000
D = 128
L = 16
NC, NS = 2, 16
NW = NC * NS
CH = 128
NCHUNK = -(-E // (NW * CH))
EPW = NCHUNK * CH
EP = EPW * NW
NP = NS * 640
RPT = NP // NS

_f32 = jnp.float32


def _mesh():
    return plsc.VectorSubcoreMesh(core_axis_name="c", subcore_axis_name="s",
                                  num_cores=NC, num_subcores=NS)


@functools.lru_cache(maxsize=None)
def _sc_agg():
    @functools.partial(
        pl.kernel,
        out_type=jax.ShapeDtypeStruct((NC * NP, D), _f32),
        mesh=_mesh(),
        scratch_types=[
            pltpu.VMEM((CH,), jnp.int32),
            pltpu.VMEM((CH,), jnp.int32),
            pltpu.VMEM((CH, D), _f32),
            pltpu.VMEM_SHARED((NP, D), _f32),
            pltpu.SemaphoreType.DMA,
        ])
    def k(h_hbm, src_hbm, dst_hbm, acc_out, srcv, dstv, rows, acc_sp, sem):
        cid = lax.axis_index("c")
        sid = lax.axis_index("s")
        wid = sid * NC + cid
        r0 = sid * RPT
        zero16 = jnp.zeros((L,), _f32)

        @pl.loop(0, CH)
        def _zero(r):
            for c in range(D // L):
                rows[r, pl.ds(c * L, L)] = zero16

        for j in range(RPT // CH):
            pltpu.sync_copy(rows, acc_sp.at[pl.ds(r0 + j * CH, CH)])
        plsc.subcore_barrier()

        ebase = wid * EPW

        @pl.loop(0, NCHUNK)
        def _edges(j):
            base = pl.multiple_of(ebase + j * CH, CH)
            pltpu.sync_copy(src_hbm.at[pl.ds(base, CH)], srcv)
            pltpu.sync_copy(dst_hbm.at[pl.ds(base, CH)], dstv)
            pltpu.async_copy(h_hbm.at[srcv], rows, sem).wait()
            pltpu.sync_copy(rows, acc_sp.at[dstv], add=True)

        plsc.subcore_barrier()
        for j in range(RPT // CH):
            row = r0 + j * CH
            pltpu.sync_copy(acc_sp.at[pl.ds(row, CH)],
                            acc_out.at[pl.ds(cid * NP + row, CH)])

    return k


@functools.lru_cache(maxsize=None)
def _sc_cnt():
    @functools.partial(
        pl.kernel,
        out_type=jax.ShapeDtypeStruct((NC * NP, D), _f32),
        mesh=_mesh(),
        scratch_types=[
            pltpu.VMEM((CH,), jnp.int32),
            pltpu.VMEM((CH, D), _f32),
            pltpu.VMEM_SHARED((NP, D), _f32),
        ])
    def k(dst_hbm, cnt_out, dstv, ones_v, cnt_sp):
        cid = lax.axis_index("c")
        sid = lax.axis_index("s")
        wid = sid * NC + cid
        r0 = sid * RPT
        zero16 = jnp.zeros((L,), _f32)
        one16 = jnp.ones((L,), _f32)

        @pl.loop(0, CH)
        def _zero(r):
            for c in range(D // L):
                ones_v[r, pl.ds(c * L, L)] = zero16

        for j in range(RPT // CH):
            pltpu.sync_copy(ones_v, cnt_sp.at[pl.ds(r0 + j * CH, CH)])

        @pl.loop(0, CH)
        def _ones(r):
            for c in range(D // L):
                ones_v[r, pl.ds(c * L, L)] = one16

        plsc.subcore_barrier()

        ebase = wid * EPW

        @pl.loop(0, NCHUNK)
        def _edges(j):
            base = pl.multiple_of(ebase + j * CH, CH)
            pltpu.sync_copy(dst_hbm.at[pl.ds(base, CH)], dstv)
            pltpu.sync_copy(ones_v, cnt_sp.at[dstv], add=True)

        plsc.subcore_barrier()
        for j in range(RPT // CH):
            row = r0 + j * CH
            pltpu.sync_copy(cnt_sp.at[pl.ds(row, CH)],
                            cnt_out.at[pl.ds(cid * NP + row, CH)])

    return k


_RB = 512


def _tc_body(x_ref, acc_ref, cnt_ref, wl_ref, wr_ref, b_ref, o_ref):
    s = acc_ref[0] + acc_ref[1]
    mean = s / jnp.maximum(cnt_ref[...], 1.0)
    out = (jnp.dot(mean, wl_ref[...], preferred_element_type=_f32)
           + jnp.dot(x_ref[...], wr_ref[...], preferred_element_type=_f32)
           + b_ref[...])
    nrm = jnp.sqrt(jnp.sum(out * out, axis=-1, keepdims=True))
    out = out / jnp.maximum(nrm, 1e-12)
    o_ref[...] = jnp.maximum(out, 0.0)


_tc_layer = pl.pallas_call(
    _tc_body,
    grid=(NP // _RB,),
    in_specs=[
        pl.BlockSpec((_RB, D), lambda i: (i, 0)),
        pl.BlockSpec((NC, _RB, D), lambda i: (0, i, 0)),
        pl.BlockSpec((_RB, D), lambda i: (i, 0)),
        pl.BlockSpec((D, D), lambda i: (0, 0)),
        pl.BlockSpec((D, D), lambda i: (0, 0)),
        pl.BlockSpec((1, D), lambda i: (0, 0)),
    ],
    out_specs=pl.BlockSpec((_RB, D), lambda i: (i, 0)),
    out_shape=jax.ShapeDtypeStruct((NP, D), _f32),
)


def kernel(x, edge_index, Wl0, Wr0, b0, Wl1, Wr1, b1, Wl2, Wr2, b2):
    src = jnp.concatenate(
        [edge_index[0], jnp.zeros((EP - E,), jnp.int32)])
    dst = jnp.concatenate(
        [edge_index[1], jnp.full((EP - E,), NP - 1, jnp.int32)])
    xp = jnp.zeros((NP, D), _f32).at[:N].set(x)

    sc_agg = _sc_agg()
    sc_cnt = _sc_cnt()

    cntp = sc_cnt(dst)
    cnt = cntp[:NP] + cntp[NP:]

    acc = sc_agg(xp, src, dst).reshape(NC, NP, D)
    h = _tc_layer(xp, acc, cnt, Wl0, Wr0, b0.reshape(1, D))
    acc = sc_agg(h, src, dst).reshape(NC, NP, D)
    h = _tc_layer(h, acc, cnt, Wl1, Wr1, b1.reshape(1, D))
    acc = sc_agg(h, src, dst).reshape(NC, NP, D)
    h = _tc_layer(h, acc, cnt, Wl2, Wr2, b2.reshape(1, D))
    return h[:N]
